# baseline (speedup 1.0000x reference)
import functools

import jax
import jax.numpy as jnp
from jax.experimental import pallas as pl
from jax.experimental.pallas import tpu as pltpu

NUM_EXPERTS = 64
TOP_K = 8
D_MODEL = 4096
TOKEN_BLOCK = 512


def _router_body(x_ref, wt_ref, probs_ref, wts_ref, idx_ref):
    logits = jnp.dot(x_ref[...], wt_ref[...], preferred_element_type=jnp.float32)
    m = jnp.max(logits, axis=-1, keepdims=True)
    e = jnp.exp(logits - m)
    probs_ref[...] = e / jnp.sum(e, axis=-1, keepdims=True)

    iota = jax.lax.broadcasted_iota(jnp.int32, logits.shape, 1)
    work = logits
    vals, idxs = [], []
    for _ in range(TOP_K):
        mj = jnp.max(work, axis=-1, keepdims=True)
        ij = jnp.min(jnp.where(work == mj, iota, NUM_EXPERTS), axis=-1, keepdims=True)
        vals.append(mj)
        idxs.append(ij)
        work = jnp.where(iota == ij, -jnp.inf, work)
    tv = jnp.concatenate(vals, axis=-1)
    ti = jnp.concatenate(idxs, axis=-1)
    ew = jnp.exp(tv - tv[:, 0:1])
    wts_ref[...] = ew / jnp.sum(ew, axis=-1, keepdims=True)
    idx_ref[...] = ti


@functools.partial(jax.jit, static_argnames=("interpret",))
def kernel(x, W_gate, interpret=False):
    b, s, d = x.shape
    n = b * s
    x2 = x.reshape(n, d)
    wt = W_gate.T
    grid = (n // TOKEN_BLOCK,)
    probs, wts, idx = pl.pallas_call(
        _router_body,
        grid=grid,
        in_specs=[
            pl.BlockSpec((TOKEN_BLOCK, d), lambda i: (i, 0)),
            pl.BlockSpec((d, NUM_EXPERTS), lambda i: (0, 0)),
        ],
        out_specs=[
            pl.BlockSpec((TOKEN_BLOCK, NUM_EXPERTS), lambda i: (i, 0)),
            pl.BlockSpec((TOKEN_BLOCK, TOP_K), lambda i: (i, 0)),
            pl.BlockSpec((TOKEN_BLOCK, TOP_K), lambda i: (i, 0)),
        ],
        out_shape=[
            jax.ShapeDtypeStruct((n, NUM_EXPERTS), jnp.float32),
            jax.ShapeDtypeStruct((n, TOP_K), jnp.float32),
            jax.ShapeDtypeStruct((n, TOP_K), jnp.int32),
        ],
        compiler_params=pltpu.CompilerParams(
            dimension_semantics=("arbitrary",),
        ),
        interpret=interpret,
    )(x2, wt)
    return (
        wts.reshape(b, s, TOP_K),
        idx.reshape(b, s, TOP_K),
        probs.reshape(b, s, NUM_EXPERTS),
    )


# speedup vs baseline: 1.0717x; 1.0717x over previous
import functools

import jax
import jax.numpy as jnp
from jax.experimental import pallas as pl
from jax.experimental.pallas import tpu as pltpu

NUM_EXPERTS = 64
TOP_K = 8
D_MODEL = 4096
TOKEN_BLOCK = 1024


def _router_body(x_ref, wt_ref, probs_ref, wts_ref, idx_ref):
    logits = jnp.dot(x_ref[...], wt_ref[...], preferred_element_type=jnp.float32)
    m = jnp.max(logits, axis=-1, keepdims=True)
    e = jnp.exp(logits - m)
    probs_ref[...] = e / jnp.sum(e, axis=-1, keepdims=True)

    iota = jax.lax.broadcasted_iota(jnp.int32, logits.shape, 1)
    work = logits
    vals, idxs = [], []
    for _ in range(TOP_K):
        mj = jnp.max(work, axis=-1, keepdims=True)
        ij = jnp.min(jnp.where(work == mj, iota, NUM_EXPERTS), axis=-1, keepdims=True)
        vals.append(mj)
        idxs.append(ij)
        work = jnp.where(iota == ij, -jnp.inf, work)
    tv = jnp.concatenate(vals, axis=-1)
    ti = jnp.concatenate(idxs, axis=-1)
    ew = jnp.exp(tv - tv[:, 0:1])
    wts_ref[...] = ew / jnp.sum(ew, axis=-1, keepdims=True)
    idx_ref[...] = ti


@functools.partial(jax.jit, static_argnames=("interpret",))
def kernel(x, W_gate, interpret=False):
    b, s, d = x.shape
    n = b * s
    x2 = x.reshape(n, d)
    wt = W_gate.T
    grid = (n // TOKEN_BLOCK,)
    probs, wts, idx = pl.pallas_call(
        _router_body,
        grid=grid,
        in_specs=[
            pl.BlockSpec((TOKEN_BLOCK, d), lambda i: (i, 0)),
            pl.BlockSpec((d, NUM_EXPERTS), lambda i: (0, 0)),
        ],
        out_specs=[
            pl.BlockSpec((TOKEN_BLOCK, NUM_EXPERTS), lambda i: (i, 0)),
            pl.BlockSpec((TOKEN_BLOCK, TOP_K), lambda i: (i, 0)),
            pl.BlockSpec((TOKEN_BLOCK, TOP_K), lambda i: (i, 0)),
        ],
        out_shape=[
            jax.ShapeDtypeStruct((n, NUM_EXPERTS), jnp.float32),
            jax.ShapeDtypeStruct((n, TOP_K), jnp.float32),
            jax.ShapeDtypeStruct((n, TOP_K), jnp.int32),
        ],
        compiler_params=pltpu.CompilerParams(
            dimension_semantics=("arbitrary",),
        ),
        interpret=interpret,
    )(x2, wt)
    return (
        wts.reshape(b, s, TOP_K),
        idx.reshape(b, s, TOP_K),
        probs.reshape(b, s, NUM_EXPERTS),
    )


# speedup vs baseline: 1.2771x; 1.1916x over previous
import functools

import jax
import jax.numpy as jnp
from jax import lax
from jax.experimental import pallas as pl
from jax.experimental.pallas import tpu as pltpu
from jax.experimental.pallas import tpu_sc as plsc

NUM_EXPERTS = 64
TOP_K = 8
D_MODEL = 4096
N_TOKENS = 16384
TOKEN_BLOCK = 1024

_NUM_WORKERS = 32
_TOK_PER_W = N_TOKENS // _NUM_WORKERS
_LANES = 16
_GROUPS = _TOK_PER_W // _LANES


def _tc_body(x_ref, wt_ref, logits_t_ref, probs_ref):
    logits = jnp.dot(x_ref[...], wt_ref[...], preferred_element_type=jnp.float32)
    logits_t_ref[...] = logits.T
    m = jnp.max(logits, axis=-1, keepdims=True)
    e = jnp.exp(logits - m)
    probs_ref[...] = e / jnp.sum(e, axis=-1, keepdims=True)


def _tc_call(x2, wt):
    n = x2.shape[0]
    grid = (n // TOKEN_BLOCK,)
    return pl.pallas_call(
        _tc_body,
        grid=grid,
        in_specs=[
            pl.BlockSpec((TOKEN_BLOCK, D_MODEL), lambda i: (i, 0)),
            pl.BlockSpec((D_MODEL, NUM_EXPERTS), lambda i: (0, 0)),
        ],
        out_specs=[
            pl.BlockSpec((NUM_EXPERTS, TOKEN_BLOCK), lambda i: (0, i)),
            pl.BlockSpec((TOKEN_BLOCK, NUM_EXPERTS), lambda i: (i, 0)),
        ],
        out_shape=[
            jax.ShapeDtypeStruct((NUM_EXPERTS, n), jnp.float32),
            jax.ShapeDtypeStruct((n, NUM_EXPERTS), jnp.float32),
        ],
        compiler_params=pltpu.CompilerParams(
            dimension_semantics=("arbitrary",),
        ),
    )(x2, wt)


def _sc_topk(logits_t_hbm, wts_t_hbm, idx_t_hbm, lg_v, wts_v, idx_v):
    wid = lax.axis_index("s") * 2 + lax.axis_index("c")
    base = wid * _TOK_PER_W
    pltpu.sync_copy(logits_t_hbm.at[:, pl.ds(base, _TOK_PER_W)], lg_v)

    def group_body(g, carry):
        g16 = g * _LANES
        neg_inf = jnp.full((_LANES,), -jnp.inf, jnp.float32)
        zero_i = jnp.zeros((_LANES,), jnp.int32)
        tv = [neg_inf] * TOP_K
        ti = [zero_i] * TOP_K
        for e in range(NUM_EXPERTS):
            col = jnp.full((_LANES,), e, jnp.int32)
            v = lg_v[e, pl.ds(g16, _LANES)]
            pred = [v > tv[j] for j in range(TOP_K)]
            ntv, nti = [], []
            for j in range(TOP_K):
                if j == 0:
                    shift_v, shift_i = v, col
                else:
                    shift_v = jnp.where(pred[j - 1], tv[j - 1], v)
                    shift_i = jnp.where(pred[j - 1], ti[j - 1], col)
                ntv.append(jnp.where(pred[j], shift_v, tv[j]))
                nti.append(jnp.where(pred[j], shift_i, ti[j]))
            tv, ti = ntv, nti
        ew = [jnp.exp(tv[j] - tv[0]) for j in range(TOP_K)]
        s = ew[0]
        for j in range(1, TOP_K):
            s = s + ew[j]
        inv = 1.0 / s
        for j in range(TOP_K):
            wts_v[j, pl.ds(g16, _LANES)] = ew[j] * inv
            idx_v[j, pl.ds(g16, _LANES)] = ti[j]
        return carry

    lax.fori_loop(0, _GROUPS, group_body, 0)

    pltpu.sync_copy(wts_v, wts_t_hbm.at[:, pl.ds(base, _TOK_PER_W)])
    pltpu.sync_copy(idx_v, idx_t_hbm.at[:, pl.ds(base, _TOK_PER_W)])


_sc_call = functools.partial(
    pl.kernel,
    out_type=[
        jax.ShapeDtypeStruct((TOP_K, N_TOKENS), jnp.float32),
        jax.ShapeDtypeStruct((TOP_K, N_TOKENS), jnp.int32),
    ],
    mesh=plsc.VectorSubcoreMesh(core_axis_name="c", subcore_axis_name="s"),
    scratch_types=[
        pltpu.VMEM((NUM_EXPERTS, _TOK_PER_W), jnp.float32),
        pltpu.VMEM((TOP_K, _TOK_PER_W), jnp.float32),
        pltpu.VMEM((TOP_K, _TOK_PER_W), jnp.int32),
    ],
)(_sc_topk)


@jax.jit
def kernel(x, W_gate):
    b, s, d = x.shape
    n = b * s
    x2 = x.reshape(n, d)
    wt = W_gate.T
    logits_t, probs = _tc_call(x2, wt)
    wts_t, idx_t = _sc_call(logits_t)
    return (
        wts_t.T.reshape(b, s, TOP_K),
        idx_t.T.reshape(b, s, TOP_K),
        probs.reshape(b, s, NUM_EXPERTS),
    )
